# baseline (device time: 94267 ns/iter reference)
import jax
import jax.numpy as jnp
from jax import lax
from jax.experimental import pallas as pl
from jax.experimental.pallas import tpu as pltpu

N_DEV = 4
N_LAYERS = 3


def kernel(x, Win0, Wout0, Win1, Wout1, Win2, Wout2):
    b, _ = x.shape
    h_dim = Win0.shape[1]
    chunk = b // N_DEV

    def body(x_ref, win0_ref, wout0_ref, win1_ref, wout1_ref, win2_ref,
             wout2_ref, out_ref, acc_ref, h_ref, rs_buf,
             rs_send_sem, rs_recv_sems, ag_send_sem, ag_recv_sems):
        my = lax.axis_index("i")
        left = lax.rem(my + N_DEV - 1, N_DEV)
        right = lax.rem(my + 1, N_DEV)

        barrier_sem = pltpu.get_barrier_semaphore()
        for nbr in (left, right):
            pl.semaphore_signal(
                barrier_sem, inc=1,
                device_id=(nbr,), device_id_type=pl.DeviceIdType.MESH,
            )
        pl.semaphore_wait(barrier_sem, 2)

        win_refs = [win0_ref, win1_ref, win2_ref]
        wout_refs = [wout0_ref, wout1_ref, wout2_ref]

        x_val = x_ref[...]
        for layer in range(N_LAYERS):
            acc_ref[...] = jnp.dot(
                x_val, win_refs[layer][...],
                preferred_element_type=jnp.float32,
            )

            for s in range(N_DEV - 1):
                send_c = lax.rem(my - s + N_DEV, N_DEV)
                rdma = pltpu.make_async_remote_copy(
                    src_ref=acc_ref.at[pl.ds(send_c * chunk, chunk), :],
                    dst_ref=rs_buf.at[s],
                    send_sem=rs_send_sem,
                    recv_sem=rs_recv_sems.at[layer, s],
                    device_id=(right,),
                    device_id_type=pl.DeviceIdType.MESH,
                )
                rdma.start()
                rdma.wait()
                recv_c = lax.rem(my - s - 1 + N_DEV, N_DEV)
                sl = pl.ds(recv_c * chunk, chunk)
                acc_ref[sl, :] = acc_ref[sl, :] + rs_buf[s]

            mine = lax.rem(my + 1, N_DEV)
            msl = pl.ds(mine * chunk, chunk)
            h_ref[msl, :] = jnp.maximum(acc_ref[msl, :], 0.0)

            for s in range(N_DEV - 1):
                send_c = lax.rem(my + 1 - s + N_DEV, N_DEV)
                ssl = pl.ds(send_c * chunk, chunk)
                rdma = pltpu.make_async_remote_copy(
                    src_ref=h_ref.at[ssl, :],
                    dst_ref=h_ref.at[ssl, :],
                    send_sem=ag_send_sem,
                    recv_sem=ag_recv_sems.at[layer, s],
                    device_id=(right,),
                    device_id_type=pl.DeviceIdType.MESH,
                )
                rdma.start()
                rdma.wait()

            x_val = jnp.dot(
                h_ref[...], wout_refs[layer][...],
                preferred_element_type=jnp.float32,
            )

        out_ref[...] = x_val

    return pl.pallas_call(
        body,
        out_shape=jax.ShapeDtypeStruct(x.shape, jnp.float32),
        in_specs=[pl.BlockSpec(memory_space=pltpu.VMEM)] * 7,
        out_specs=pl.BlockSpec(memory_space=pltpu.VMEM),
        scratch_shapes=[
            pltpu.VMEM((b, h_dim), jnp.float32),
            pltpu.VMEM((b, h_dim), jnp.float32),
            pltpu.VMEM((N_DEV - 1, chunk, h_dim), jnp.float32),
            pltpu.SemaphoreType.DMA,
            pltpu.SemaphoreType.DMA((N_LAYERS, N_DEV - 1)),
            pltpu.SemaphoreType.DMA,
            pltpu.SemaphoreType.DMA((N_LAYERS, N_DEV - 1)),
        ],
        compiler_params=pltpu.CompilerParams(collective_id=0),
    )(x, Win0, Wout0, Win1, Wout1, Win2, Wout2)


# device time: 58232 ns/iter; 1.6188x vs baseline; 1.6188x over previous
import jax
import jax.numpy as jnp
from jax import lax
from jax.experimental import pallas as pl
from jax.experimental.pallas import tpu as pltpu

N_DEV = 4
N_LAYERS = 3

RS1A, RS1B, RS2A, RS2B, AG1A, AG1B, AG2A, AG2B = range(8)


def kernel(x, Win0, Wout0, Win1, Wout1, Win2, Wout2):
    b, _ = x.shape
    h_dim = Win0.shape[1]
    half = b // 2
    sub = b // 4
    qtr = b // 8

    def body(x_ref, win0_ref, wout0_ref, win1_ref, wout1_ref, win2_ref,
             wout2_ref, out_ref, acc_ref, h_ref, xbuf_ref, rs1_buf, rs2_buf,
             send_sems, recv_sems):
        my = lax.axis_index("i")
        pa = my ^ 1
        pb = 3 - my
        g = (my ^ (my >> 1)) & 1
        ra = my & 1
        rb = my >> 1

        kA = g * sub
        sA = (1 - g) * sub
        kB = half + rb * sub
        sB = half + (1 - rb) * sub
        qA = kA + rb * qtr
        sqA = kA + (1 - rb) * qtr
        qB = kB + ra * qtr
        sqB = kB + (1 - ra) * qtr

        barrier_sem = pltpu.get_barrier_semaphore()
        for nbr in (pa, pb):
            pl.semaphore_signal(
                barrier_sem, inc=1,
                device_id=(nbr,), device_id_type=pl.DeviceIdType.MESH,
            )
        pl.semaphore_wait(barrier_sem, 2)

        def exch(slot, src_ref, src_start, nrows, dst_ref, dst_start, peer):
            rdma = pltpu.make_async_remote_copy(
                src_ref=src_ref.at[pl.ds(src_start, nrows), :],
                dst_ref=dst_ref.at[pl.ds(dst_start, nrows), :],
                send_sem=send_sems.at[slot],
                recv_sem=recv_sems.at[slot],
                device_id=(peer,),
                device_id_type=pl.DeviceIdType.MESH,
            )
            rdma.start()
            return rdma

        def finish(*rdmas):
            for r in rdmas:
                r.wait_recv()
            for r in rdmas:
                r.wait_send()

        win_refs = [win0_ref, win1_ref, win2_ref]
        wout_refs = [wout0_ref, wout1_ref, wout2_ref]

        for layer in range(N_LAYERS):
            xsrc = x_ref if layer == 0 else xbuf_ref
            xdst = out_ref if layer == N_LAYERS - 1 else xbuf_ref
            win = win_refs[layer]
            wout = wout_refs[layer]

            def blk_dot(dst, rows_start, nrows, src, w):
                dst[pl.ds(rows_start, nrows), :] = jnp.dot(
                    src[pl.ds(rows_start, nrows), :], w[...],
                    preferred_element_type=jnp.float32,
                )

            blk_dot(acc_ref, sA, sub, xsrc, win)
            blk_dot(acc_ref, sB, sub, xsrc, win)

            r1a = exch(RS1A, acc_ref, sA, sub, rs1_buf.at[0], 0, pa)
            r1b = exch(RS1B, acc_ref, sB, sub, rs1_buf.at[1], 0, pb)
            blk_dot(acc_ref, kA, sub, xsrc, win)
            blk_dot(acc_ref, kB, sub, xsrc, win)
            finish(r1a, r1b)
            slA = pl.ds(kA, sub)
            slB = pl.ds(kB, sub)
            acc_ref[slA, :] = acc_ref[slA, :] + rs1_buf[0]
            acc_ref[slB, :] = acc_ref[slB, :] + rs1_buf[1]

            r2a = exch(RS2A, acc_ref, sqA, qtr, rs2_buf.at[0], 0, pb)
            r2b = exch(RS2B, acc_ref, sqB, qtr, rs2_buf.at[1], 0, pa)
            finish(r2a, r2b)
            qlA = pl.ds(qA, qtr)
            qlB = pl.ds(qB, qtr)
            h_ref[qlA, :] = jnp.maximum(acc_ref[qlA, :] + rs2_buf[0], 0.0)
            h_ref[qlB, :] = jnp.maximum(acc_ref[qlB, :] + rs2_buf[1], 0.0)

            a1a = exch(AG1A, h_ref, qA, qtr, h_ref, qA, pb)
            a1b = exch(AG1B, h_ref, qB, qtr, h_ref, qB, pa)
            finish(a1a, a1b)

            a2a = exch(AG2A, h_ref, kA, sub, h_ref, kA, pa)
            a2b = exch(AG2B, h_ref, kB, sub, h_ref, kB, pb)
            blk_dot(xdst, kA, sub, h_ref, wout)
            blk_dot(xdst, kB, sub, h_ref, wout)
            finish(a2a, a2b)
            blk_dot(xdst, sA, sub, h_ref, wout)
            blk_dot(xdst, sB, sub, h_ref, wout)

    return pl.pallas_call(
        body,
        out_shape=jax.ShapeDtypeStruct(x.shape, jnp.float32),
        in_specs=[pl.BlockSpec(memory_space=pltpu.VMEM)] * 7,
        out_specs=pl.BlockSpec(memory_space=pltpu.VMEM),
        scratch_shapes=[
            pltpu.VMEM((b, h_dim), jnp.float32),
            pltpu.VMEM((b, h_dim), jnp.float32),
            pltpu.VMEM(x.shape, jnp.float32),
            pltpu.VMEM((2, sub, h_dim), jnp.float32),
            pltpu.VMEM((2, qtr, h_dim), jnp.float32),
            pltpu.SemaphoreType.DMA((8,)),
            pltpu.SemaphoreType.DMA((8,)),
        ],
        compiler_params=pltpu.CompilerParams(collective_id=0),
    )(x, Win0, Wout0, Win1, Wout1, Win2, Wout2)


# device time: 53740 ns/iter; 1.7541x vs baseline; 1.0836x over previous
import jax
import jax.numpy as jnp
from jax import lax
from jax.experimental import pallas as pl
from jax.experimental.pallas import tpu as pltpu

N_DEV = 4
N_LAYERS = 3

S1A, S1B, S2A, S2B, S3A, S3B = range(6)


def kernel(x, Win0, Wout0, Win1, Wout1, Win2, Wout2):
    b, _ = x.shape
    h_dim = Win0.shape[1]
    half = b // 2
    sub = b // 4

    def body(x_ref, win0_ref, wout0_ref, win1_ref, wout1_ref, win2_ref,
             wout2_ref, out_ref, acc_ref, h_ref, xbuf_ref, s1_buf, s2_buf,
             send_sems, recv_sems):
        my = lax.axis_index("i")
        pa = my ^ 1
        pb = 3 - my
        g = (my ^ (my >> 1)) & 1
        rb = my >> 1

        kA = g * sub
        sA = (1 - g) * sub
        kB = half + rb * sub
        sB = half + (1 - rb) * sub

        barrier_sem = pltpu.get_barrier_semaphore()
        for nbr in (pa, pb):
            pl.semaphore_signal(
                barrier_sem, inc=1,
                device_id=(nbr,), device_id_type=pl.DeviceIdType.MESH,
            )
        pl.semaphore_wait(barrier_sem, 2)

        def exch(slot, src_ref, src_start, dst_ref, dst_start, peer):
            rdma = pltpu.make_async_remote_copy(
                src_ref=src_ref.at[pl.ds(src_start, sub), :],
                dst_ref=dst_ref.at[pl.ds(dst_start, sub), :],
                send_sem=send_sems.at[slot],
                recv_sem=recv_sems.at[slot],
                device_id=(peer,),
                device_id_type=pl.DeviceIdType.MESH,
            )
            rdma.start()
            return rdma

        def finish(*rdmas):
            for r in rdmas:
                r.wait_recv()
            for r in rdmas:
                r.wait_send()

        win_refs = [win0_ref, win1_ref, win2_ref]
        wout_refs = [wout0_ref, wout1_ref, wout2_ref]

        for layer in range(N_LAYERS):
            xsrc = x_ref if layer == 0 else xbuf_ref
            xdst = out_ref if layer == N_LAYERS - 1 else xbuf_ref
            win = win_refs[layer]
            wout = wout_refs[layer]

            def blk_dot(dst, rows_start, src, w):
                dst[pl.ds(rows_start, sub), :] = jnp.dot(
                    src[pl.ds(rows_start, sub), :], w[...],
                    preferred_element_type=jnp.float32,
                )

            blk_dot(acc_ref, sA, xsrc, win)
            blk_dot(acc_ref, sB, xsrc, win)

            r1a = exch(S1A, acc_ref, sA, s1_buf.at[0], 0, pa)
            r1b = exch(S1B, acc_ref, sB, s1_buf.at[1], 0, pb)
            blk_dot(acc_ref, kA, xsrc, win)
            blk_dot(acc_ref, kB, xsrc, win)
            finish(r1a, r1b)
            slA = pl.ds(kA, sub)
            slB = pl.ds(kB, sub)
            acc_ref[slA, :] = acc_ref[slA, :] + s1_buf[0]
            acc_ref[slB, :] = acc_ref[slB, :] + s1_buf[1]

            r2a = exch(S2A, acc_ref, kA, s2_buf.at[0], 0, pb)
            r2b = exch(S2B, acc_ref, kB, s2_buf.at[1], 0, pa)
            finish(r2a, r2b)
            h_ref[slA, :] = jnp.maximum(acc_ref[slA, :] + s2_buf[0], 0.0)
            h_ref[slB, :] = jnp.maximum(acc_ref[slB, :] + s2_buf[1], 0.0)

            r3a = exch(S3A, h_ref, kA, h_ref, kA, pa)
            r3b = exch(S3B, h_ref, kB, h_ref, kB, pb)
            blk_dot(xdst, kA, h_ref, wout)
            blk_dot(xdst, kB, h_ref, wout)
            finish(r3a, r3b)
            blk_dot(xdst, sA, h_ref, wout)
            blk_dot(xdst, sB, h_ref, wout)

    return pl.pallas_call(
        body,
        out_shape=jax.ShapeDtypeStruct(x.shape, jnp.float32),
        in_specs=[pl.BlockSpec(memory_space=pltpu.VMEM)] * 7,
        out_specs=pl.BlockSpec(memory_space=pltpu.VMEM),
        scratch_shapes=[
            pltpu.VMEM((b, h_dim), jnp.float32),
            pltpu.VMEM((b, h_dim), jnp.float32),
            pltpu.VMEM(x.shape, jnp.float32),
            pltpu.VMEM((2, sub, h_dim), jnp.float32),
            pltpu.VMEM((2, sub, h_dim), jnp.float32),
            pltpu.SemaphoreType.DMA((6,)),
            pltpu.SemaphoreType.DMA((6,)),
        ],
        compiler_params=pltpu.CompilerParams(collective_id=0),
    )(x, Win0, Wout0, Win1, Wout1, Win2, Wout2)


# device time: 45572 ns/iter; 2.0685x vs baseline; 1.1792x over previous
import jax
import jax.numpy as jnp
from jax import lax
from jax.experimental import pallas as pl
from jax.experimental.pallas import tpu as pltpu

N_DEV = 4
N_LAYERS = 3
N_COL = 2

def _slot(stage, half, col):
    return (stage * 2 + half) * N_COL + col

N_SLOTS = 3 * 2 * N_COL


def kernel(x, Win0, Wout0, Win1, Wout1, Win2, Wout2):
    b, _ = x.shape
    h_dim = Win0.shape[1]
    half = b // 2
    sub = b // 4
    cw = h_dim // N_COL

    def body(x_ref, win0_ref, wout0_ref, win1_ref, wout1_ref, win2_ref,
             wout2_ref, out_ref, acc_ref, h_ref, xbuf_ref, s1_buf, s2_buf,
             send_sems, recv_sems):
        my = lax.axis_index("i")
        pa = my ^ 1
        pb = 3 - my
        g = (my ^ (my >> 1)) & 1
        rb = my >> 1

        kA = g * sub
        sA = (1 - g) * sub
        kB = half + rb * sub
        sB = half + (1 - rb) * sub
        keep = (kA, kB)
        sent = (sA, sB)
        p1 = (pa, pb)
        p2 = (pb, pa)

        barrier_sem = pltpu.get_barrier_semaphore()
        for nbr in (pa, pb):
            pl.semaphore_signal(
                barrier_sem, inc=1,
                device_id=(nbr,), device_id_type=pl.DeviceIdType.MESH,
            )
        pl.semaphore_wait(barrier_sem, 2)

        def exch(stage, hf, col, src_ref, src_start, dst_ref, dst_start, peer):
            sl = _slot(stage, hf, col)
            rdma = pltpu.make_async_remote_copy(
                src_ref=src_ref.at[pl.ds(src_start, sub),
                                   pl.ds(col * cw, cw)],
                dst_ref=dst_ref.at[pl.ds(dst_start, sub),
                                   pl.ds(col * cw, cw)],
                send_sem=send_sems.at[sl],
                recv_sem=recv_sems.at[sl],
                device_id=(peer,),
                device_id_type=pl.DeviceIdType.MESH,
            )
            rdma.start()
            return rdma

        win_refs = [win0_ref, win1_ref, win2_ref]
        wout_refs = [wout0_ref, wout1_ref, wout2_ref]

        for layer in range(N_LAYERS):
            xsrc = x_ref if layer == 0 else xbuf_ref
            xdst = out_ref if layer == N_LAYERS - 1 else xbuf_ref
            win = win_refs[layer]
            wout = wout_refs[layer]

            def dot_tile(dst, rows, col):
                dst[pl.ds(rows, sub), pl.ds(col * cw, cw)] = jnp.dot(
                    xsrc[pl.ds(rows, sub), :], win[:, pl.ds(col * cw, cw)],
                    preferred_element_type=jnp.float32,
                )

            r1 = {}
            for col in range(N_COL):
                for hf in range(2):
                    dot_tile(acc_ref, sent[hf], col)
                    r1[hf, col] = exch(0, hf, col, acc_ref, sent[hf],
                                       s1_buf.at[hf], 0, p1[hf])
            for col in range(N_COL):
                for hf in range(2):
                    dot_tile(acc_ref, keep[hf], col)

            r2 = {}
            for col in range(N_COL):
                for hf in range(2):
                    r1[hf, col].wait_recv()
                    rows = pl.ds(keep[hf], sub)
                    cols = pl.ds(col * cw, cw)
                    acc_ref[rows, cols] = (
                        acc_ref[rows, cols] + s1_buf[hf, :, col * cw:(col + 1) * cw]
                    )
                    r2[hf, col] = exch(1, hf, col, acc_ref, keep[hf],
                                       s2_buf.at[hf], 0, p2[hf])
            r3 = {}
            for col in range(N_COL):
                for hf in range(2):
                    r2[hf, col].wait_recv()
                    rows = pl.ds(keep[hf], sub)
                    cols = pl.ds(col * cw, cw)
                    h_ref[rows, cols] = jnp.maximum(
                        acc_ref[rows, cols] + s2_buf[hf, :, col * cw:(col + 1) * cw],
                        0.0,
                    )
                    r3[hf, col] = exch(2, hf, col, h_ref, keep[hf],
                                       h_ref, keep[hf], p1[hf])

            for hf in range(2):
                xdst[pl.ds(keep[hf], sub), :] = jnp.dot(
                    h_ref[pl.ds(keep[hf], sub), :], wout[...],
                    preferred_element_type=jnp.float32,
                )
            for hf in range(2):
                for col in range(N_COL):
                    r3[hf, col].wait_recv()
                xdst[pl.ds(sent[hf], sub), :] = jnp.dot(
                    h_ref[pl.ds(sent[hf], sub), :], wout[...],
                    preferred_element_type=jnp.float32,
                )
            for r in list(r1.values()) + list(r2.values()) + list(r3.values()):
                r.wait_send()

    return pl.pallas_call(
        body,
        out_shape=jax.ShapeDtypeStruct(x.shape, jnp.float32),
        in_specs=[pl.BlockSpec(memory_space=pltpu.VMEM)] * 7,
        out_specs=pl.BlockSpec(memory_space=pltpu.VMEM),
        scratch_shapes=[
            pltpu.VMEM((b, h_dim), jnp.float32),
            pltpu.VMEM((b, h_dim), jnp.float32),
            pltpu.VMEM(x.shape, jnp.float32),
            pltpu.VMEM((2, sub, h_dim), jnp.float32),
            pltpu.VMEM((2, sub, h_dim), jnp.float32),
            pltpu.SemaphoreType.DMA((N_SLOTS,)),
            pltpu.SemaphoreType.DMA((N_SLOTS,)),
        ],
        compiler_params=pltpu.CompilerParams(collective_id=0),
    )(x, Win0, Wout0, Win1, Wout1, Win2, Wout2)


# device time: 44119 ns/iter; 2.1367x vs baseline; 1.0329x over previous
import jax
import jax.numpy as jnp
from jax import lax
from jax.experimental import pallas as pl
from jax.experimental.pallas import tpu as pltpu

N_DEV = 4
N_LAYERS = 3
N_COL = 4

def _slot(stage, half, col):
    return (stage * 2 + half) * N_COL + col

N_SLOTS = 3 * 2 * N_COL


def kernel(x, Win0, Wout0, Win1, Wout1, Win2, Wout2):
    b, _ = x.shape
    h_dim = Win0.shape[1]
    half = b // 2
    sub = b // 4
    cw = h_dim // N_COL

    def body(x_ref, win0_ref, wout0_ref, win1_ref, wout1_ref, win2_ref,
             wout2_ref, out_ref, acc_ref, h_ref, xbuf_ref, s1_buf, s2_buf,
             send_sems, recv_sems):
        my = lax.axis_index("i")
        pa = my ^ 1
        pb = 3 - my
        g = (my ^ (my >> 1)) & 1
        rb = my >> 1

        kA = g * sub
        sA = (1 - g) * sub
        kB = half + rb * sub
        sB = half + (1 - rb) * sub
        keep = (kA, kB)
        sent = (sA, sB)
        p1 = (pa, pb)
        p2 = (pb, pa)

        barrier_sem = pltpu.get_barrier_semaphore()
        for nbr in (pa, pb):
            pl.semaphore_signal(
                barrier_sem, inc=1,
                device_id=(nbr,), device_id_type=pl.DeviceIdType.MESH,
            )
        pl.semaphore_wait(barrier_sem, 2)

        def exch(stage, hf, col, src_ref, src_start, dst_ref, dst_start, peer):
            sl = _slot(stage, hf, col)
            rdma = pltpu.make_async_remote_copy(
                src_ref=src_ref.at[pl.ds(src_start, sub),
                                   pl.ds(col * cw, cw)],
                dst_ref=dst_ref.at[pl.ds(dst_start, sub),
                                   pl.ds(col * cw, cw)],
                send_sem=send_sems.at[sl],
                recv_sem=recv_sems.at[sl],
                device_id=(peer,),
                device_id_type=pl.DeviceIdType.MESH,
            )
            rdma.start()
            return rdma

        win_refs = [win0_ref, win1_ref, win2_ref]
        wout_refs = [wout0_ref, wout1_ref, wout2_ref]

        for layer in range(N_LAYERS):
            xsrc = x_ref if layer == 0 else xbuf_ref
            xdst = out_ref if layer == N_LAYERS - 1 else xbuf_ref
            win = win_refs[layer]
            wout = wout_refs[layer]

            def dot_tile(dst, rows, col):
                dst[pl.ds(rows, sub), pl.ds(col * cw, cw)] = jnp.dot(
                    xsrc[pl.ds(rows, sub), :], win[:, pl.ds(col * cw, cw)],
                    preferred_element_type=jnp.float32,
                )

            r1 = {}
            for col in range(N_COL):
                for hf in range(2):
                    dot_tile(acc_ref, sent[hf], col)
                    r1[hf, col] = exch(0, hf, col, acc_ref, sent[hf],
                                       s1_buf.at[hf], 0, p1[hf])
            for col in range(N_COL):
                for hf in range(2):
                    dot_tile(acc_ref, keep[hf], col)

            r2 = {}
            for col in range(N_COL):
                for hf in range(2):
                    r1[hf, col].wait_recv()
                    rows = pl.ds(keep[hf], sub)
                    cols = pl.ds(col * cw, cw)
                    acc_ref[rows, cols] = (
                        acc_ref[rows, cols] + s1_buf[hf, :, col * cw:(col + 1) * cw]
                    )
                    r2[hf, col] = exch(1, hf, col, acc_ref, keep[hf],
                                       s2_buf.at[hf], 0, p2[hf])
            r3 = {}
            for col in range(N_COL):
                for hf in range(2):
                    r2[hf, col].wait_recv()
                    rows = pl.ds(keep[hf], sub)
                    cols = pl.ds(col * cw, cw)
                    h_ref[rows, cols] = jnp.maximum(
                        acc_ref[rows, cols] + s2_buf[hf, :, col * cw:(col + 1) * cw],
                        0.0,
                    )
                    r3[hf, col] = exch(2, hf, col, h_ref, keep[hf],
                                       h_ref, keep[hf], p1[hf])

            for hf in range(2):
                xdst[pl.ds(keep[hf], sub), :] = jnp.dot(
                    h_ref[pl.ds(keep[hf], sub), :], wout[...],
                    preferred_element_type=jnp.float32,
                )
            for hf in range(2):
                for col in range(N_COL):
                    r3[hf, col].wait_recv()
                xdst[pl.ds(sent[hf], sub), :] = jnp.dot(
                    h_ref[pl.ds(sent[hf], sub), :], wout[...],
                    preferred_element_type=jnp.float32,
                )
            for r in list(r1.values()) + list(r2.values()) + list(r3.values()):
                r.wait_send()

    return pl.pallas_call(
        body,
        out_shape=jax.ShapeDtypeStruct(x.shape, jnp.float32),
        in_specs=[pl.BlockSpec(memory_space=pltpu.VMEM)] * 7,
        out_specs=pl.BlockSpec(memory_space=pltpu.VMEM),
        scratch_shapes=[
            pltpu.VMEM((b, h_dim), jnp.float32),
            pltpu.VMEM((b, h_dim), jnp.float32),
            pltpu.VMEM(x.shape, jnp.float32),
            pltpu.VMEM((2, sub, h_dim), jnp.float32),
            pltpu.VMEM((2, sub, h_dim), jnp.float32),
            pltpu.SemaphoreType.DMA((N_SLOTS,)),
            pltpu.SemaphoreType.DMA((N_SLOTS,)),
        ],
        compiler_params=pltpu.CompilerParams(collective_id=0),
    )(x, Win0, Wout0, Win1, Wout1, Win2, Wout2)


# device time: 35184 ns/iter; 2.6793x vs baseline; 1.2540x over previous
import jax
import jax.numpy as jnp
from jax import lax
from jax.experimental import pallas as pl
from jax.experimental.pallas import tpu as pltpu

N_DEV = 4
N_LAYERS = 3
N_COL = 4

def _slot(stage, half, col):
    return (stage * 2 + half) * N_COL + col

N_SLOTS = 3 * 2 * N_COL


def kernel(x, Win0, Wout0, Win1, Wout1, Win2, Wout2):
    b, _ = x.shape
    h_dim = Win0.shape[1]
    half = b // 2
    sub = b // 4
    cw = h_dim // N_COL

    def body(x_ref, win0_ref, wout0_ref, win1_ref, wout1_ref, win2_ref,
             wout2_ref, out_ref, acc_ref, h_ref, xbuf_ref, hbf_ref,
             tx1_buf, tx2_buf, s1_buf, s2_buf, send_sems, recv_sems):
        my = lax.axis_index("i")
        pa = my ^ 1
        pb = 3 - my
        g = (my ^ (my >> 1)) & 1
        rb = my >> 1

        kA = g * sub
        sA = (1 - g) * sub
        kB = half + rb * sub
        sB = half + (1 - rb) * sub
        keep = (kA, kB)
        sent = (sA, sB)
        p1 = (pa, pb)
        p2 = (pb, pa)

        barrier_sem = pltpu.get_barrier_semaphore()
        for nbr in (pa, pb):
            pl.semaphore_signal(
                barrier_sem, inc=1,
                device_id=(nbr,), device_id_type=pl.DeviceIdType.MESH,
            )
        pl.semaphore_wait(barrier_sem, 2)

        def exch(stage, hf, col, src_ref, dst_ref, peer):
            sl = _slot(stage, hf, col)
            rdma = pltpu.make_async_remote_copy(
                src_ref=src_ref,
                dst_ref=dst_ref,
                send_sem=send_sems.at[sl],
                recv_sem=recv_sems.at[sl],
                device_id=(peer,),
                device_id_type=pl.DeviceIdType.MESH,
            )
            rdma.start()
            return rdma

        win_refs = [win0_ref, win1_ref, win2_ref]
        wout_refs = [wout0_ref, wout1_ref, wout2_ref]

        for layer in range(N_LAYERS):
            xsrc = x_ref if layer == 0 else xbuf_ref
            xdst = out_ref if layer == N_LAYERS - 1 else xbuf_ref
            win = win_refs[layer]
            wout = wout_refs[layer]

            r1 = {}
            for col in range(N_COL):
                cols = pl.ds(col * cw, cw)
                for hf in range(2):
                    tx1_buf[hf, :, cols] = jnp.dot(
                        xsrc[pl.ds(sent[hf], sub), :], win[:, cols],
                        preferred_element_type=jnp.float32,
                    ).astype(jnp.bfloat16)
                    r1[hf, col] = exch(
                        0, hf, col,
                        tx1_buf.at[hf, :, cols], s1_buf.at[hf, :, cols],
                        p1[hf],
                    )
            for col in range(N_COL):
                cols = pl.ds(col * cw, cw)
                for hf in range(2):
                    acc_ref[pl.ds(keep[hf], sub), cols] = jnp.dot(
                        xsrc[pl.ds(keep[hf], sub), :], win[:, cols],
                        preferred_element_type=jnp.float32,
                    )

            r2 = {}
            for col in range(N_COL):
                cols = pl.ds(col * cw, cw)
                for hf in range(2):
                    r1[hf, col].wait_recv()
                    rows = pl.ds(keep[hf], sub)
                    summed = acc_ref[rows, cols] + s1_buf[hf, :, cols].astype(
                        jnp.float32
                    )
                    acc_ref[rows, cols] = summed
                    tx2_buf[hf, :, cols] = summed.astype(jnp.bfloat16)
                    r2[hf, col] = exch(
                        1, hf, col,
                        tx2_buf.at[hf, :, cols], s2_buf.at[hf, :, cols],
                        p2[hf],
                    )
            r3 = {}
            for col in range(N_COL):
                cols = pl.ds(col * cw, cw)
                for hf in range(2):
                    r2[hf, col].wait_recv()
                    rows = pl.ds(keep[hf], sub)
                    hred = jnp.maximum(
                        acc_ref[rows, cols] + s2_buf[hf, :, cols].astype(
                            jnp.float32
                        ),
                        0.0,
                    )
                    h_ref[rows, cols] = hred
                    hbf_ref[rows, cols] = hred.astype(jnp.bfloat16)
                    r3[hf, col] = exch(
                        2, hf, col,
                        hbf_ref.at[rows, cols], hbf_ref.at[rows, cols],
                        p1[hf],
                    )

            for hf in range(2):
                xdst[pl.ds(keep[hf], sub), :] = jnp.dot(
                    h_ref[pl.ds(keep[hf], sub), :], wout[...],
                    preferred_element_type=jnp.float32,
                )
            for col in range(N_COL):
                cols = pl.ds(col * cw, cw)
                for hf in range(2):
                    r3[hf, col].wait_recv()
                    rows = pl.ds(sent[hf], sub)
                    contrib = jnp.dot(
                        hbf_ref[rows, cols].astype(jnp.float32),
                        wout[cols, :],
                        preferred_element_type=jnp.float32,
                    )
                    if col == 0:
                        xdst[rows, :] = contrib
                    else:
                        xdst[rows, :] = xdst[rows, :] + contrib
            for r in list(r1.values()) + list(r2.values()) + list(r3.values()):
                r.wait_send()

    return pl.pallas_call(
        body,
        out_shape=jax.ShapeDtypeStruct(x.shape, jnp.float32),
        in_specs=[pl.BlockSpec(memory_space=pltpu.VMEM)] * 7,
        out_specs=pl.BlockSpec(memory_space=pltpu.VMEM),
        scratch_shapes=[
            pltpu.VMEM((b, h_dim), jnp.float32),
            pltpu.VMEM((b, h_dim), jnp.float32),
            pltpu.VMEM(x.shape, jnp.float32),
            pltpu.VMEM((b, h_dim), jnp.bfloat16),
            pltpu.VMEM((2, sub, h_dim), jnp.bfloat16),
            pltpu.VMEM((2, sub, h_dim), jnp.bfloat16),
            pltpu.VMEM((2, sub, h_dim), jnp.bfloat16),
            pltpu.VMEM((2, sub, h_dim), jnp.bfloat16),
            pltpu.SemaphoreType.DMA((N_SLOTS,)),
            pltpu.SemaphoreType.DMA((N_SLOTS,)),
        ],
        compiler_params=pltpu.CompilerParams(collective_id=0),
    )(x, Win0, Wout0, Win1, Wout1, Win2, Wout2)
